# baseline (speedup 1.0000x reference)
import functools

import jax
import jax.numpy as jnp
from jax import lax
from jax.experimental import pallas as pl
from jax.experimental.pallas import tpu as pltpu
from jax.experimental.pallas import tpu_sc as plsc

B = 4096
L = 20
D = 32
H1 = 128
H2 = 64

NC = 2
NS = 16
NW = NC * NS
SPW = B // NW
RPW = SPW * L
CH = 128
NCHUNK = RPW // CH


def _pool_body(tq_hbm, tc_hbm, qidx_hbm, cidx_hbm, outq_hbm, outc_hbm,
               idx_v, rows_v, pooled_v, sem):
    wid = lax.axis_index("s") * NC + lax.axis_index("c")
    base_s = wid * SPW
    base_i = wid * RPW

    for tower in range(2):
        tbl = tq_hbm if tower == 0 else tc_hbm
        sidx = qidx_hbm if tower == 0 else cidx_hbm
        out = outq_hbm if tower == 0 else outc_hbm

        pltpu.sync_copy(sidx.at[pl.ds(base_i, RPW)], idx_v)

        copies = []
        for j in range(NCHUNK):
            copies.append(pltpu.async_copy(
                tbl.at[idx_v.at[pl.ds(j * CH, CH)]],
                rows_v.at[pl.ds(j * CH, CH)],
                sem))
        for c in copies:
            c.wait()

        def pool_one(s, carry):
            r0 = s * L
            a0 = rows_v[r0, 0:16]
            a1 = rows_v[r0, 16:32]
            for l in range(1, L):
                a0 = a0 + rows_v[r0 + l, 0:16]
                a1 = a1 + rows_v[r0 + l, 16:32]
            pooled_v[s, 0:16] = a0
            pooled_v[s, 16:32] = a1
            return carry

        lax.fori_loop(0, SPW, pool_one, 0, unroll=2)

        pltpu.sync_copy(pooled_v, out.at[pl.ds(base_s, SPW)])


def _pooled_sc(table_q, table_c, qidx_flat, cidx_flat):
    mesh = plsc.VectorSubcoreMesh(core_axis_name="c", subcore_axis_name="s")
    return pl.kernel(
        _pool_body,
        out_type=(
            jax.ShapeDtypeStruct((B, D), jnp.float32),
            jax.ShapeDtypeStruct((B, D), jnp.float32),
        ),
        mesh=mesh,
        scratch_types=[
            pltpu.VMEM((RPW,), jnp.int32),
            pltpu.VMEM((RPW, D), jnp.float32),
            pltpu.VMEM((SPW, D), jnp.float32),
            pltpu.SemaphoreType.DMA,
        ],
        compiler_params=pltpu.CompilerParams(use_tc_tiling_on_sc=False),
    )(table_q, table_c, qidx_flat, cidx_flat)


def _mlp_body(xq_ref, xc_ref, wq1_ref, bq1_ref, wq2_ref, bq2_ref,
              wc1_ref, bc1_ref, wc2_ref, bc2_ref, oq_ref, oc_ref):
    hq = jnp.dot(xq_ref[...], wq1_ref[...], preferred_element_type=jnp.float32)
    hq = jnp.maximum(hq + bq1_ref[...], 0.0)
    oq = jnp.dot(hq, wq2_ref[...], preferred_element_type=jnp.float32)
    oq_ref[...] = jnp.maximum(oq + bq2_ref[...], 0.0)

    hc = jnp.dot(xc_ref[...], wc1_ref[...], preferred_element_type=jnp.float32)
    hc = jnp.maximum(hc + bc1_ref[...], 0.0)
    oc = jnp.dot(hc, wc2_ref[...], preferred_element_type=jnp.float32)
    oc_ref[...] = jnp.maximum(oc + bc2_ref[...], 0.0)


def _mlp_tc(pooled_q, pooled_c, Wq1, bq1, Wq2, bq2, Wc1, bc1, Wc2, bc2):
    BLK = 512
    grid = (B // BLK,)
    full = lambda r, c: pl.BlockSpec((r, c), lambda i: (0, 0))
    return pl.pallas_call(
        _mlp_body,
        grid=grid,
        in_specs=[
            pl.BlockSpec((BLK, D), lambda i: (i, 0)),
            pl.BlockSpec((BLK, D), lambda i: (i, 0)),
            full(D, H1), full(1, H1), full(H1, H2), full(1, H2),
            full(D, H1), full(1, H1), full(H1, H2), full(1, H2),
        ],
        out_specs=[
            pl.BlockSpec((BLK, H2), lambda i: (i, 0)),
            pl.BlockSpec((BLK, H2), lambda i: (i, 0)),
        ],
        out_shape=[
            jax.ShapeDtypeStruct((B, H2), jnp.float32),
            jax.ShapeDtypeStruct((B, H2), jnp.float32),
        ],
    )(pooled_q, pooled_c, Wq1, bq1, Wq2, bq2, Wc1, bc1, Wc2, bc2)


def kernel(query_indices, candidate_indices, table_q, table_c,
           Wq1, bq1, Wq2, bq2, Wc1, bc1, Wc2, bc2):
    qidx_flat = query_indices.astype(jnp.int32).reshape(B * L)
    cidx_flat = candidate_indices.astype(jnp.int32).reshape(B * L)

    pooled_q, pooled_c = _pooled_sc(table_q, table_c, qidx_flat, cidx_flat)

    q, c = _mlp_tc(pooled_q, pooled_c,
                   Wq1, bq1[None, :], Wq2, bq2[None, :],
                   Wc1, bc1[None, :], Wc2, bc2[None, :])
    return q, c
